# baseline (speedup 1.0000x reference)
import functools
import math

import jax
import jax.numpy as jnp
from jax import lax
from jax.experimental import pallas as pl
from jax.experimental.pallas import tpu as pltpu
from jax.experimental.pallas import tpu_sc as plsc

_B, _L, _D = 4096, 200, 64
_NTB = 64
_TWO_PI = 2.0 * math.pi

_B1 = 3072
_SB = _B - _B1
_NW = 32
_CH = 400


def _bucket_level(delta, f32, bitcast=lax.bitcast_convert_type):
    dm1 = jnp.maximum(delta.astype(f32), 1.0) / 60.0 + 1.0
    ebits = bitcast(dm1, jnp.int32)
    bidx = jnp.clip((ebits >> 23) - 127, 0, _NTB - 1)
    level = (delta > 1800).astype(jnp.int32) + (delta > 86400).astype(jnp.int32)
    return bidx, level


def _prep_body(tt_ref, ss_ref, wp_ref, bp_ref, wf_ref, bf_ref, ts_ref, mc_ref):
    f32 = jnp.float32
    wf = wf_ref[...]
    dn = (((1,), (1,)), ((), ()))
    tf = lax.dot_general(tt_ref[...], wf[:, 0:64], dn, preferred_element_type=f32)
    sf = lax.dot_general(ss_ref[...], wf[:, 64:128], dn, preferred_element_type=f32)
    m = lax.dot_general(wp_ref[...], wf[:, 128:192], (((0,), (1,)), ((), ())),
                        preferred_element_type=f32)
    c = lax.dot_general(bp_ref[...], wf[:, 128:192], dn, preferred_element_type=f32)
    c = c + bf_ref[...]
    ts = (lax.broadcast_in_dim(tf, (64, 3, _D), (0, 2))
          + lax.broadcast_in_dim(sf[0:3, :], (64, 3, _D), (1, 2))
          + lax.broadcast_in_dim(c, (64, 3, _D), (1, 2)))
    ts_ref[...] = ts.reshape(192, _D)
    mc_ref[...] = m


def _prep_tables(time_table, sess8, Wp, bp2, Wf, bf2):
    full = lambda shape: pl.BlockSpec(shape, lambda: (0, 0))
    return pl.pallas_call(
        _prep_body,
        in_specs=[full((_NTB, _D)), full((8, _D)), full((_D, 4)),
                  full((1, _D)), full((_D, 3 * _D)), full((1, _D))],
        out_specs=[full((192, _D)), full((4, _D))],
        out_shape=[jax.ShapeDtypeStruct((192, _D), jnp.float32),
                   jax.ShapeDtypeStruct((4, _D), jnp.float32)],
    )(time_table, sess8, Wp, bp2, Wf, bf2)


def _tc_body(seq_ref, cur_ref, tt_ref, ss_ref, wp_ref, bp_ref, wf_ref, bf_ref,
             out_ref, *, tb):
    f32 = jnp.float32
    wf = wf_ref[...]
    dn = (((1,), (1,)), ((), ()))
    tf = lax.dot_general(tt_ref[...], wf[:, 0:64], dn, preferred_element_type=f32)
    sf = lax.dot_general(ss_ref[...], wf[:, 64:128], dn, preferred_element_type=f32)
    m = lax.dot_general(wp_ref[...], wf[:, 128:192], (((0,), (1,)), ((), ())),
                        preferred_element_type=f32)
    m8 = jnp.concatenate([m, jnp.zeros((4, _D), f32)], axis=0)
    c = lax.dot_general(bp_ref[...], wf[:, 128:192], dn, preferred_element_type=f32)
    c = c + bf_ref[...]

    seq = seq_ref[...]
    cur = cur_ref[...]
    delta = jnp.maximum(cur - seq, 0)
    bidx, level = _bucket_level(delta, f32)

    hour_i = lax.rem(seq, 86400)
    a1 = hour_i.astype(f32) * f32(_TWO_PI / 86400.0)
    day = lax.rem(seq.astype(f32) / 86400.0, 7.0)
    a2 = day * f32(_TWO_PI / 7.0)

    def pack2(x, y):
        xb = lax.bitcast_convert_type(x, jnp.int32)
        yb = lax.bitcast_convert_type(y, jnp.int32)
        xr = (xb + 0x8000) & jnp.int32(0xFFFF0000 - 0x100000000)
        return xr | ((yb + 0x8000) >> 16) & 0xFFFF

    p1 = pack2(jnp.sin(a1), jnp.cos(a1))
    p2 = pack2(jnp.sin(a2), jnp.cos(a2))

    n = tb * _L

    def minor(x, k):
        return lax.broadcast_in_dim(x, (tb, _L, k), (0, 1))

    io64 = lax.broadcasted_iota(jnp.int32, (tb, _L, _NTB), 2)
    oh_t = (io64 == minor(bidx, _NTB)).astype(f32).reshape(n, _NTB)
    io8 = lax.broadcasted_iota(jnp.int32, (tb, _L, 8), 2)
    oh_s = (io8 == minor(level, 8)).astype(f32).reshape(n, 8)
    p1b = minor(p1, 8)
    p2b = minor(p2, 8)

    def hi(p):
        return lax.bitcast_convert_type(p & jnp.int32(0xFFFF0000 - 0x100000000), f32)

    def lo(p):
        return lax.bitcast_convert_type(p << 16, f32)

    zero = jnp.zeros((), f32)
    per = (jnp.where(io8 == 0, hi(p1b), zero)
           + jnp.where(io8 == 1, lo(p1b), zero)
           + jnp.where(io8 == 2, hi(p2b), zero)
           + jnp.where(io8 == 3, lo(p2b), zero)).reshape(n, 8)

    acc = lax.dot_general(oh_t, tf, (((1,), (0,)), ((), ())),
                          preferred_element_type=f32)
    acc = acc + lax.dot_general(oh_s, sf, (((1,), (0,)), ((), ())),
                                preferred_element_type=f32)
    acc = acc + lax.dot_general(per, m8, (((1,), (0,)), ((), ())),
                                preferred_element_type=f32)
    out_ref[...] = acc + c


def _tc_part(seq1, cur2, time_table, sess8, Wp, bp2, Wf, bf2):
    tb = 128
    grid = _B1 // tb
    full = lambda shape: pl.BlockSpec(shape, lambda i: (0, 0))
    out_flat = pl.pallas_call(
        functools.partial(_tc_body, tb=tb),
        grid=(grid,),
        in_specs=[
            pl.BlockSpec((tb, _L), lambda i: (i, 0)),
            pl.BlockSpec((tb, 1), lambda i: (i, 0)),
            full((_NTB, _D)),
            full((8, _D)),
            full((_D, 4)),
            full((1, _D)),
            full((_D, 3 * _D)),
            full((1, _D)),
        ],
        out_specs=pl.BlockSpec((tb * _L, _D), lambda i: (i, 0)),
        out_shape=jax.ShapeDtypeStruct((_B * _L, _D), jnp.float32),
        compiler_params=pltpu.CompilerParams(
            dimension_semantics=("parallel",)),
    )(seq1, cur2, time_table, sess8, Wp, bp2, Wf, bf2)
    return out_flat.reshape(_B, _L, _D)


def _sincos(x, f32):
    u4 = x * f32(2.0 / math.pi)
    q = u4.astype(jnp.int32)
    th = (u4 - q.astype(f32)) * f32(math.pi / 2.0)
    w = th * th
    s0 = th * (1.0 + w * (f32(-1.0 / 6.0)
                          + w * (f32(1.0 / 120.0) + w * f32(-1.0 / 5040.0))))
    c0 = 1.0 + w * (f32(-0.5) + w * (f32(1.0 / 24.0)
                                     + w * (f32(-1.0 / 720.0)
                                            + w * f32(1.0 / 40320.0))))
    qodd = (q & 1) == 1
    sabs = jnp.where(qodd, c0, s0)
    sinx = jnp.where(q >= 2, -sabs, sabs)
    cabs = jnp.where(qodd, s0, c0)
    cosx = jnp.where((q == 1) | (q == 2), -cabs, cabs)
    return sinx, cosx


def _sc_group(seq_v, curb_v, tsc_v, mc_v, out_v, ci_s, ps_s, cbase, g):
    f32 = jnp.float32
    off = g * 16
    s = seq_v[pl.ds(off, 16)]
    cu = curb_v[pl.ds(off, 16)]
    delta = jnp.maximum(cu - s, 0)
    bidx, level = _bucket_level(delta, f32, bitcast=plsc.bitcast)
    ci = bidx * 3 + level

    q0 = (s.astype(f32) * f32(1.0 / 86400.0)).astype(jnp.int32)
    r = s - q0 * 86400
    r = r + jnp.where(r < 0, 86400, 0)
    r = r - jnp.where(r >= 86400, 86400, 0)
    a1 = r.astype(f32) * f32(_TWO_PI / 86400.0)
    d = s.astype(f32) / f32(86400.0)
    day = d - 7.0 * (d * f32(1.0 / 7.0)).astype(jnp.int32).astype(f32)
    a2 = day * f32(_TWO_PI / 7.0)
    s1, c1 = _sincos(a1, f32)
    s2, c2 = _sincos(a2, f32)

    iota16 = lax.iota(jnp.int32, 16)
    ci_s[pl.ds(16, 16)] = ci
    ps_s[pl.ds(16, 16)] = s1
    ps_s[pl.ds(32, 16)] = c1
    ps_s[pl.ds(48, 16)] = s2
    ps_s[pl.ds(64, 16)] = c2

    for e in range(16):
        cib = plsc.load_gather(ci_s, [jnp.full((16,), 16 + e, jnp.int32)])
        p1 = plsc.load_gather(ps_s, [jnp.full((16,), 16 + e, jnp.int32)])
        p2 = plsc.load_gather(ps_s, [jnp.full((16,), 32 + e, jnp.int32)])
        p3 = plsc.load_gather(ps_s, [jnp.full((16,), 48 + e, jnp.int32)])
        p4 = plsc.load_gather(ps_s, [jnp.full((16,), 64 + e, jnp.int32)])
        gbase = (cib << 6)
        for dc in range(4):
            gidx = gbase + (iota16 + dc * 16)
            row = plsc.load_gather(tsc_v, [gidx])
            acc = (row + p1 * mc_v[pl.ds(0 * _D + dc * 16, 16)]
                   + p2 * mc_v[pl.ds(1 * _D + dc * 16, 16)]
                   + p3 * mc_v[pl.ds(2 * _D + dc * 16, 16)]
                   + p4 * mc_v[pl.ds(3 * _D + dc * 16, 16)])
            out_v[pl.ds(off * _D + e * _D + dc * 16, 16)] = acc


def _sc_part(seq_flat, curb_flat, tsc_flat, mc_flat):
    mesh = plsc.VectorSubcoreMesh(core_axis_name="c", subcore_axis_name="s")
    per_w = _SB * _L // _NW
    nchunks = per_w // _CH
    ngroups = _CH // 16

    @functools.partial(
        pl.kernel, mesh=mesh,
        compiler_params=pltpu.CompilerParams(needs_layout_passes=False),
        out_type=jax.ShapeDtypeStruct((_SB * _L * _D,), jnp.float32),
        scratch_types=[
            pltpu.VMEM((_CH,), jnp.int32),
            pltpu.VMEM((_CH,), jnp.int32),
            pltpu.VMEM((192 * _D,), jnp.float32),
            pltpu.VMEM((4 * _D,), jnp.float32),
            pltpu.VMEM((_CH * _D,), jnp.float32),
            pltpu.VMEM((32,), jnp.int32),
            pltpu.VMEM((80,), jnp.float32),
        ],
    )
    def k(seq_hbm, curb_hbm, tsc_hbm, mc_hbm, out_hbm,
          seq_v, curb_v, tsc_v, mc_v, out_v, ci_s, ps_s):
        wid = lax.axis_index("s") * 2 + lax.axis_index("c")
        base = wid * per_w
        pltpu.sync_copy(tsc_hbm, tsc_v)
        pltpu.sync_copy(mc_hbm, mc_v)

        def chunk_body(ch, carry):
            cb = base + ch * _CH
            pltpu.sync_copy(seq_hbm.at[pl.ds(cb, _CH)], seq_v)
            pltpu.sync_copy(curb_hbm.at[pl.ds(cb, _CH)], curb_v)

            def group_body(g, carry2):
                _sc_group(seq_v, curb_v, tsc_v, mc_v, out_v, ci_s, ps_s, cb, g)
                return carry2

            lax.fori_loop(0, ngroups, group_body, 0)
            pltpu.sync_copy(out_v, out_hbm.at[pl.ds(cb * _D, _CH * _D)])
            return carry

        lax.fori_loop(0, nchunks, chunk_body, 0)

    return k(seq_flat, curb_flat, tsc_flat, mc_flat)


def kernel(seq_timestamps, current_timestamp, time_table, sess_table, Wp, bp, Wf, bf):
    cur2 = current_timestamp.reshape(_B, 1)
    sess8 = jnp.concatenate(
        [sess_table, jnp.zeros((8 - sess_table.shape[0], _D), jnp.float32)], axis=0)
    bp2 = bp.reshape(1, _D)
    bf2 = bf.reshape(1, _D)

    ts, mc = _prep_tables(time_table, sess8, Wp, bp2, Wf, bf2)
    tsc_flat = ts.reshape(192 * _D)
    mc_flat = mc.reshape(4 * _D)

    out_tc = _tc_part(seq_timestamps, cur2, time_table, sess8, Wp, bp2, Wf, bf2)

    seq_sc = seq_timestamps[_B1:, :].reshape(_SB * _L)
    curb_sc = jnp.broadcast_to(current_timestamp[_B1:, None],
                               (_SB, _L)).reshape(_SB * _L)
    out_sc = _sc_part(seq_sc, curb_sc, tsc_flat, mc_flat).reshape(_SB, _L, _D)

    return lax.dynamic_update_slice(out_tc, out_sc, (_B1, 0, 0))
